# baseline (speedup 1.0000x reference)
import functools

import jax
import jax.numpy as jnp
from jax import lax
from jax.experimental import pallas as pl
from jax.experimental.pallas import tpu as pltpu
from jax.experimental.pallas import tpu_sc as plsc

B = 4096
F = 26
D_DENSE = 13
FEAT = 4000
DIM = 16
TOTAL = F * FEAT
WCOL = F * DIM
ROW = 512

NC = 2
NS = 16
NW = NC * NS

ARC = 64
ATOT = TOTAL // ARC
ANC = -(-ATOT // NW)

NB = B // NW
CHUNK = 4
NCHUNK = NB // CHUNK
ROWS_PER_CHUNK = CHUNK * F


def _sc_build_table(embed_tables, w16r, eye16):
    mesh = plsc.VectorSubcoreMesh(core_axis_name="c", subcore_axis_name="s")

    @functools.partial(
        pl.kernel,
        mesh=mesh,
        out_type=jax.ShapeDtypeStruct((TOTAL, ROW), jnp.float32),
        scratch_types=[
            pltpu.VMEM((8, ARC // 8, 128), jnp.float32),
            pltpu.VMEM((8, DIM), jnp.float32),
            pltpu.VMEM((DIM, DIM), jnp.float32),
            pltpu.VMEM((ARC, ROW), jnp.float32),
            pltpu.SemaphoreType.DMA,
            pltpu.SemaphoreType.DMA,
        ],
    )
    def k(emb_hbm, w_hbm, eye_hbm, t_hbm, fbuf_v, wslab_v, eye_v, abuf_v,
          sem_r, sem_w):
        wid = lax.axis_index("s") * NC + lax.axis_index("c")
        pltpu.sync_copy(eye_hbm, eye_v)

        def zero_body(i, carry):
            for g in range(WCOL // DIM + 1, ROW // DIM):
                abuf_v[i, pl.ds(g * DIM, DIM)] = jnp.zeros((DIM,),
                                                           jnp.float32)
            return carry

        lax.fori_loop(0, ARC, zero_body, 0, unroll=False)

        def slab_body(c, carry):
            idx = c * NW + wid

            @pl.when(idx < ATOT)
            def _():
                r0 = idx * ARC
                wcp = pltpu.async_copy(
                    w_hbm.at[pl.ds((idx // 2) * 8, 8), :], wslab_v, sem_w)
                v0 = idx * (ARC // 8)
                copies = [pltpu.async_copy(
                    emb_hbm.at[j, pl.ds(v0, ARC // 8), :], fbuf_v.at[j],
                    sem_r) for j in range(8)]
                for j in range(F):
                    copies[j].wait()
                    if j + 8 < F:
                        copies.append(pltpu.async_copy(
                            emb_hbm.at[j + 8, pl.ds(v0, ARC // 8), :],
                            fbuf_v.at[(j + 8) % 8], sem_r))
                    p = j % 8

                    def copy_body(vr, carry2, p=p, j=j):
                        for u in range(8):
                            abuf_v[vr * 8 + u, pl.ds(j * DIM, DIM)] = \
                                fbuf_v[p, vr, pl.ds(u * DIM, DIM)]
                        return carry2

                    lax.fori_loop(0, ARC // 8, copy_body, 0, unroll=False)
                wcp.wait()
                woff = (idx % 2) * (ARC // DIM)

                def w_body(q, carry2):
                    for u in range(DIM):
                        abuf_v[q * DIM + u, pl.ds(WCOL, DIM)] = (
                            wslab_v[woff + q, :] * eye_v[u, :])
                    return carry2

                lax.fori_loop(0, ARC // DIM, w_body, 0, unroll=False)
                pltpu.async_copy(
                    abuf_v, t_hbm.at[pl.ds(r0, ARC), :], sem_w).wait()

            return carry

        lax.fori_loop(0, ANC, slab_body, 0, unroll=False)

    return k(embed_tables, w16r, eye16)


def _sc_gather_cross(table, sp_flat):
    mesh = plsc.VectorSubcoreMesh(core_axis_name="c", subcore_axis_name="s")

    @functools.partial(
        pl.kernel,
        mesh=mesh,
        out_type=jax.ShapeDtypeStruct((B, DIM), jnp.float32),
        scratch_types=[
            pltpu.VMEM((NB * F,), jnp.int32),
            pltpu.VMEM((ROWS_PER_CHUNK, ROW), jnp.float32),
            pltpu.VMEM((NB, DIM), jnp.float32),
            pltpu.SemaphoreType.DMA,
        ],
    )
    def k(table_hbm, sp_hbm, out_hbm, idx_v, rows_v, out_v, sem):
        wid = lax.axis_index("s") * NC + lax.axis_index("c")
        base = wid * (NB * F)
        pltpu.sync_copy(sp_hbm.at[pl.ds(base, NB * F)], idx_v)

        def chunk_body(c, carry):
            pltpu.async_copy(
                table_hbm.at[idx_v.at[pl.ds(c * ROWS_PER_CHUNK,
                                            ROWS_PER_CHUNK)]],
                rows_v, sem).wait()

            def b_body(bb, carry2):
                r0 = bb * F
                acc = jnp.zeros((DIM,), jnp.float32)
                for i in range(F - 1):
                    for j in range(i + 1, F):
                        acc = acc + (rows_v[r0 + i, pl.ds(j * DIM, DIM)] *
                                     rows_v[r0 + j, pl.ds(i * DIM, DIM)])
                for i in range(F):
                    acc = acc + rows_v[r0 + i, pl.ds(WCOL, DIM)]
                out_v[c * CHUNK + bb, :] = acc
                return carry2

            lax.fori_loop(0, CHUNK, b_body, 0, unroll=False)
            return carry

        lax.fori_loop(0, NCHUNK, chunk_body, 0, unroll=False)
        pltpu.sync_copy(out_v, out_hbm.at[pl.ds(wid * NB, NB)])

    return k(table, sp_flat)


def _final_body(dense_ref, wd_ref, b_ref, part_ref, o_ref):
    lin = jnp.sum(dense_ref[...] * wd_ref[...], axis=1, keepdims=True)
    cross = jnp.sum(part_ref[...], axis=1, keepdims=True)
    o_ref[...] = jax.nn.sigmoid(lin + cross + b_ref[0, 0])


def _final(dense, wd_row, bias11, partial):
    return pl.pallas_call(
        _final_body,
        out_shape=jax.ShapeDtypeStruct((B, 1), jnp.float32),
    )(dense, wd_row, bias11, partial)


def kernel(dense_input, sparse_input, bias, weight_dense, weight_sparse,
           embed_tables):
    offs = jnp.arange(F, dtype=jnp.int32) * FEAT
    sp_flat = (sparse_input + offs[None, :]).reshape(B * F)
    w16r = weight_sparse.reshape(TOTAL // DIM, DIM)
    eye16 = jnp.eye(DIM, dtype=jnp.float32)
    emb_view = embed_tables.reshape(F, TOTAL * DIM // 128, 128)
    table = _sc_build_table(emb_view, w16r, eye16)
    partial = _sc_gather_cross(table, sp_flat)
    return _final(dense_input, weight_dense.reshape(1, D_DENSE),
                  bias.reshape(1, 1), partial)


# speedup vs baseline: 1.3967x; 1.3967x over previous
import functools

import jax
import jax.numpy as jnp
from jax import lax
from jax.experimental import pallas as pl
from jax.experimental.pallas import tpu as pltpu
from jax.experimental.pallas import tpu_sc as plsc

B = 4096
F = 26
D_DENSE = 13
FEAT = 4000
DIM = 16
TOTAL = F * FEAT
WCOL = F * DIM
ROW = 512

NC = 2
NS = 16
NW = NC * NS
NB = B // NW
CHUNK = 4
NCHUNK = NB // CHUNK
ROWS_PER_CHUNK = CHUNK * F

BT = 1000


def _build_table_body(emb_ref, w_ref, t_ref):
    for j in range(F):
        t_ref[:, j * DIM:(j + 1) * DIM] = emb_ref[j, :, :]
    w = w_ref[0, 0, :].reshape(BT, 1)
    t_ref[:, WCOL:] = jnp.concatenate(
        [w, jnp.zeros((BT, ROW - WCOL - 1), jnp.float32)], axis=1)


def _build_table(embed_tables, weight_sparse):
    return pl.pallas_call(
        _build_table_body,
        grid=(TOTAL // BT,),
        in_specs=[
            pl.BlockSpec((F, BT, DIM), lambda t: (0, t, 0)),
            pl.BlockSpec((1, 1, BT), lambda t: (t, 0, 0)),
        ],
        out_specs=pl.BlockSpec((BT, ROW), lambda t: (t, 0)),
        out_shape=jax.ShapeDtypeStruct((TOTAL, ROW), jnp.float32),
    )(embed_tables, weight_sparse.reshape(TOTAL // BT, 1, BT))


def _sc_gather_cross(table, sp_flat):
    mesh = plsc.VectorSubcoreMesh(core_axis_name="c", subcore_axis_name="s")

    @functools.partial(
        pl.kernel,
        mesh=mesh,
        out_type=jax.ShapeDtypeStruct((B, DIM), jnp.float32),
        scratch_types=[
            pltpu.VMEM((NB * F,), jnp.int32),
            pltpu.VMEM((ROWS_PER_CHUNK, ROW), jnp.float32),
            pltpu.VMEM((NB, DIM), jnp.float32),
            pltpu.SemaphoreType.DMA,
        ],
    )
    def k(table_hbm, sp_hbm, out_hbm, idx_v, rows_v, out_v, sem):
        wid = lax.axis_index("s") * NC + lax.axis_index("c")
        base = wid * (NB * F)
        pltpu.sync_copy(sp_hbm.at[pl.ds(base, NB * F)], idx_v)

        def chunk_body(c, carry):
            pltpu.async_copy(
                table_hbm.at[idx_v.at[pl.ds(c * ROWS_PER_CHUNK,
                                            ROWS_PER_CHUNK)]],
                rows_v, sem).wait()

            def b_body(bb, carry2):
                r0 = bb * F
                acc = jnp.zeros((DIM,), jnp.float32)
                for i in range(F - 1):
                    for j in range(i + 1, F):
                        acc = acc + (rows_v[r0 + i, pl.ds(j * DIM, DIM)] *
                                     rows_v[r0 + j, pl.ds(i * DIM, DIM)])
                for i in range(F):
                    acc = acc + rows_v[r0 + i, pl.ds(WCOL, DIM)]
                out_v[c * CHUNK + bb, :] = acc
                return carry2

            lax.fori_loop(0, CHUNK, b_body, 0, unroll=False)
            return carry

        lax.fori_loop(0, NCHUNK, chunk_body, 0, unroll=False)
        pltpu.sync_copy(out_v, out_hbm.at[pl.ds(wid * NB, NB)])

    return k(table, sp_flat)


def _final_body(dense_ref, wd_ref, b_ref, part_ref, o_ref):
    lin = jnp.sum(dense_ref[...] * wd_ref[...], axis=1, keepdims=True)
    cross = jnp.sum(part_ref[...], axis=1, keepdims=True)
    o_ref[...] = jax.nn.sigmoid(lin + cross + b_ref[0, 0])


def _final(dense, wd_row, bias11, partial):
    return pl.pallas_call(
        _final_body,
        out_shape=jax.ShapeDtypeStruct((B, 1), jnp.float32),
    )(dense, wd_row, bias11, partial)


def kernel(dense_input, sparse_input, bias, weight_dense, weight_sparse,
           embed_tables):
    offs = jnp.arange(F, dtype=jnp.int32) * FEAT
    sp_flat = (sparse_input + offs[None, :]).reshape(B * F)
    table = jnp.concatenate(
        [jnp.transpose(embed_tables, (1, 0, 2)).reshape(TOTAL, WCOL),
         weight_sparse,
         jnp.zeros((TOTAL, ROW - WCOL - 1), jnp.float32)], axis=1)
    partial = _sc_gather_cross(table, sp_flat)
    return _final(dense_input, weight_dense.reshape(1, D_DENSE),
                  bias.reshape(1, 1), partial)
